# baseline (speedup 1.0000x reference)
import functools

import numpy as np

import jax
import jax.numpy as jnp
from jax import lax
from jax.experimental import pallas as pl
from jax.experimental.pallas import tpu as pltpu
from jax.experimental.pallas import tpu_sc as plsc

N_ROWS = 128
N_COLS = 100000
CHUNK = 20000
N_CHUNKS = N_COLS // CHUNK
ROWS_PER_W = 4
UNROLL = 2

_U32 = jnp.uint32
_KS1 = 42
_KS2 = 0x1BD11BDA ^ 42


def _rotl(x, r):
    return lax.shift_left(x, _U32(r)) | lax.shift_right_logical(x, _U32(32 - r))


def _threefry_bits(cnt):
    x0 = jnp.zeros_like(cnt)
    x1 = cnt + _U32(_KS1)
    rots1 = (13, 15, 26, 6)
    rots2 = (17, 29, 16, 24)
    kinj = ((_KS1, _KS2 + 1), (_KS2, 0 + 2), (0, _KS1 + 3),
            (_KS1, _KS2 + 4), (_KS2, 0 + 5))
    for gi, rots in enumerate((rots1, rots2, rots1, rots2, rots1)):
        for r in rots:
            x0 = x0 + x1
            x1 = _rotl(x1, r)
            x1 = x1 ^ x0
        ka, kb = kinj[gi]
        x0 = x0 + _U32(ka)
        x1 = x1 + _U32(kb)
    return x0 ^ x1


_LN2_HI = np.float32(0.693359375)
_LN2_LO = np.float32(-2.12194440054e-4)
_SQRT2 = np.float32(1.4142135623730951)
_TINY = np.float32(1.1754943508222875e-38)


def _neg_log(u):
    iv = lax.bitcast_convert_type(u, jnp.int32)
    e = lax.shift_right_arithmetic(iv, 23) - 127
    m = lax.bitcast_convert_type((iv & 0x7FFFFF) | 0x3F800000, jnp.float32)
    big = m > _SQRT2
    m = jnp.where(big, m * jnp.float32(0.5), m)
    e = jnp.where(big, e + 1, e)
    ef = e.astype(jnp.float32)
    z = m - jnp.float32(1.0)
    s = z / (z + jnp.float32(2.0))
    s2 = s * s
    r = jnp.float32(2.0 / 9.0)
    for c in (2.0 / 7.0, 2.0 / 5.0, 2.0 / 3.0, 2.0):
        r = r * s2 + jnp.float32(c)
    ln1pz = s * r
    return -(ef * _LN2_HI + (ef * _LN2_LO + ln1pz))


def _gumbel_ratio(cnt_u32):
    bits = _threefry_bits(cnt_u32)
    fbits = lax.shift_right_logical(bits, _U32(9)) | _U32(0x3F800000)
    f = lax.bitcast_convert_type(fbits, jnp.float32) - jnp.float32(1.0)
    u = jnp.maximum(f, _TINY)
    return _neg_log(u)


def _sampler_body(inp, out, buf, res, me, mw_r, mj_r):
    wid = lax.axis_index("s") * 2 + lax.axis_index("c")
    iota_i = lax.iota(jnp.int32, 16)
    iota_u = lax.convert_element_type(iota_i, jnp.uint32)
    res_vec = jnp.zeros((16,), jnp.int32)

    for l in range(ROWS_PER_W):
        row = wid * ROWS_PER_W + l
        row_base = row * N_COLS

        def chunk_body(c, carry, row_base=row_base):
            bt, bw, bj = carry
            off = pl.multiple_of(row_base + c * CHUNK, 8)
            pltpu.sync_copy(inp.at[pl.ds(off, CHUNK)], buf)
            cbase = c * CHUNK

            def inner(t, carry2, cbase=cbase, row_base=row_base):
                bt, bw, bj = carry2
                j0 = t * (16 * UNROLL)
                for k in range(UNROLL):
                    jk = j0 + 16 * k
                    cbits = lax.convert_element_type(row_base + cbase + jk,
                                                     jnp.uint32)
                    cnt = cbits + iota_u
                    ev = _gumbel_ratio(cnt)
                    wv = jnp.maximum(buf[pl.ds(jk, 16)], jnp.float32(1e-30))
                    pred = ev * bw[k] < bt[k] * wv
                    jv = (cbase + jk) + iota_i
                    bt[k] = jnp.where(pred, ev, bt[k])
                    bw[k] = jnp.where(pred, wv, bw[k])
                    bj[k] = jnp.where(pred, jv, bj[k])
                return bt, bw, bj

            bt, bw, bj = lax.fori_loop(
                0, CHUNK // (16 * UNROLL), inner, (bt, bw, bj))
            return bt, bw, bj

        init = ([jnp.full((16,), jnp.inf, jnp.float32)] * UNROLL,
                [jnp.ones((16,), jnp.float32)] * UNROLL,
                [jnp.zeros((16,), jnp.int32)] * UNROLL)
        bt, bw, bj = lax.fori_loop(0, N_CHUNKS, chunk_body, init)

        mt, mw, mj = bt[0], bw[0], bj[0]
        for k in range(1, UNROLL):
            t1 = bt[k] * mw
            t2 = mt * bw[k]
            p = (t1 < t2) | ((t1 == t2) & (bj[k] < mj))
            mt = jnp.where(p, bt[k], mt)
            mw = jnp.where(p, bw[k], mw)
            mj = jnp.where(p, bj[k], mj)
        for d in (8, 4, 2, 1):
            me[...] = mt
            mw_r[...] = mw
            mj_r[...] = mj
            perm = iota_i ^ d
            e2 = plsc.load_gather(me, [perm])
            w2 = plsc.load_gather(mw_r, [perm])
            j2 = plsc.load_gather(mj_r, [perm])
            t1 = e2 * mw
            t2 = mt * w2
            p = (t1 < t2) | ((t1 == t2) & (j2 < mj))
            mt = jnp.where(p, e2, mt)
            mw = jnp.where(p, w2, mw)
            mj = jnp.where(p, j2, mj)
        res_vec = jnp.where(iota_i == l, mj, res_vec)

    res[...] = res_vec
    pltpu.sync_copy(res, out.at[wid])


@functools.partial(jax.jit, static_argnums=())
def kernel(input):
    flat = input.reshape(-1)
    mesh = plsc.VectorSubcoreMesh(core_axis_name="c", subcore_axis_name="s")
    sampler = pl.kernel(
        _sampler_body,
        mesh=mesh,
        compiler_params=pltpu.CompilerParams(needs_layout_passes=False),
        out_type=jax.ShapeDtypeStruct((32, 16), jnp.int32),
        scratch_types=[
            pltpu.VMEM((CHUNK,), jnp.float32),
            pltpu.VMEM((16,), jnp.int32),
            pltpu.VMEM((16,), jnp.float32),
            pltpu.VMEM((16,), jnp.float32),
            pltpu.VMEM((16,), jnp.int32),
        ],
    )
    out32 = sampler(flat)
    return out32[:, :ROWS_PER_W].reshape(N_ROWS).astype(jnp.int64)


# speedup vs baseline: 1.1070x; 1.1070x over previous
import functools

import numpy as np

import jax
import jax.numpy as jnp
from jax import lax
from jax.experimental import pallas as pl
from jax.experimental.pallas import tpu as pltpu
from jax.experimental.pallas import tpu_sc as plsc

N_ROWS = 128
N_COLS = 100000
TC_COLS = 81920
SC_COLS = N_COLS - TC_COLS
ROWS_PER_W = 4
UNROLL = 2
TC_BR = 8
TC_BC = 1024

_U32 = jnp.uint32
_KS1 = 42
_KS2 = 0x1BD11BDA ^ 42


def _rotl(x, r):
    return lax.shift_left(x, _U32(r)) | lax.shift_right_logical(x, _U32(32 - r))


def _threefry_bits(cnt):
    x0 = jnp.zeros_like(cnt)
    x1 = cnt + _U32(_KS1)
    rots1 = (13, 15, 26, 6)
    rots2 = (17, 29, 16, 24)
    kinj = ((_KS1, _KS2 + 1), (_KS2, 0 + 2), (0, _KS1 + 3),
            (_KS1, _KS2 + 4), (_KS2, 0 + 5))
    for gi, rots in enumerate((rots1, rots2, rots1, rots2, rots1)):
        for r in rots:
            x0 = x0 + x1
            x1 = _rotl(x1, r)
            x1 = x1 ^ x0
        ka, kb = kinj[gi]
        x0 = x0 + _U32(ka)
        x1 = x1 + _U32(kb)
    return x0 ^ x1


_LN2_HI = np.float32(0.693359375)
_LN2_LO = np.float32(-2.12194440054e-4)
_SQRT2 = np.float32(1.4142135623730951)
_TINY = np.float32(1.1754943508222875e-38)


def _neg_log_poly(u):
    iv = lax.bitcast_convert_type(u, jnp.int32)
    e = lax.shift_right_arithmetic(iv, 23) - 127
    m = lax.bitcast_convert_type((iv & 0x7FFFFF) | 0x3F800000, jnp.float32)
    big = m > _SQRT2
    m = jnp.where(big, m * jnp.float32(0.5), m)
    e = jnp.where(big, e + 1, e)
    ef = e.astype(jnp.float32)
    z = m - jnp.float32(1.0)
    s = z / (z + jnp.float32(2.0))
    s2 = s * s
    r = jnp.float32(2.0 / 9.0)
    for c in (2.0 / 7.0, 2.0 / 5.0, 2.0 / 3.0, 2.0):
        r = r * s2 + jnp.float32(c)
    ln1pz = s * r
    return -(ef * _LN2_HI + (ef * _LN2_LO + ln1pz))


def _uniform(bits):
    fbits = lax.shift_right_logical(bits, _U32(9)) | _U32(0x3F800000)
    f = lax.bitcast_convert_type(fbits, jnp.float32) - jnp.float32(1.0)
    return jnp.maximum(f, _TINY)


def _sc_body(inp, out, buf, res, me, mw_r, mj_r):
    wid = lax.axis_index("s") * 2 + lax.axis_index("c")
    iota_i = lax.iota(jnp.int32, 16)
    iota_u = lax.convert_element_type(iota_i, jnp.uint32)
    res_j = jnp.zeros((16,), jnp.int32)
    res_e = jnp.zeros((16,), jnp.float32)
    res_w = jnp.zeros((16,), jnp.float32)

    for l in range(ROWS_PER_W):
        row = wid * ROWS_PER_W + l
        row_base = row * N_COLS + TC_COLS
        off = pl.multiple_of(row_base, 8)
        pltpu.sync_copy(inp.at[pl.ds(off, SC_COLS)], buf)

        def inner(t, carry, row_base=row_base):
            bt, bw, bj = carry
            j0 = t * (16 * UNROLL)
            for k in range(UNROLL):
                jk = j0 + 16 * k
                cbits = lax.convert_element_type(row_base + jk, jnp.uint32)
                cnt = cbits + iota_u
                ev = _neg_log_poly(_uniform(_threefry_bits(cnt)))
                wv = jnp.maximum(buf[pl.ds(jk, 16)], jnp.float32(1e-30))
                pred = ev * bw[k] < bt[k] * wv
                jv = (TC_COLS + jk) + iota_i
                bt[k] = jnp.where(pred, ev, bt[k])
                bw[k] = jnp.where(pred, wv, bw[k])
                bj[k] = jnp.where(pred, jv, bj[k])
            return bt, bw, bj

        init = ([jnp.full((16,), jnp.inf, jnp.float32)] * UNROLL,
                [jnp.ones((16,), jnp.float32)] * UNROLL,
                [jnp.zeros((16,), jnp.int32)] * UNROLL)
        bt, bw, bj = lax.fori_loop(0, SC_COLS // (16 * UNROLL), inner, init)

        mt, mw, mj = bt[0], bw[0], bj[0]
        for k in range(1, UNROLL):
            t1 = bt[k] * mw
            t2 = mt * bw[k]
            p = (t1 < t2) | ((t1 == t2) & (bj[k] < mj))
            mt = jnp.where(p, bt[k], mt)
            mw = jnp.where(p, bw[k], mw)
            mj = jnp.where(p, bj[k], mj)
        for d in (8, 4, 2, 1):
            me[...] = mt
            mw_r[...] = mw
            mj_r[...] = mj
            perm = iota_i ^ d
            e2 = plsc.load_gather(me, [perm])
            w2 = plsc.load_gather(mw_r, [perm])
            j2 = plsc.load_gather(mj_r, [perm])
            t1 = e2 * mw
            t2 = mt * w2
            p = (t1 < t2) | ((t1 == t2) & (j2 < mj))
            mt = jnp.where(p, e2, mt)
            mw = jnp.where(p, w2, mw)
            mj = jnp.where(p, j2, mj)
        res_j = jnp.where(iota_i == l, mj, res_j)
        res_e = jnp.where(iota_i == l, mt, res_e)
        res_w = jnp.where(iota_i == l, mw, res_w)

    me[...] = res_e
    mw_r[...] = res_w
    idx4 = iota_i & 3
    e_perm = plsc.load_gather(me, [idx4])
    w_perm = plsc.load_gather(mw_r, [idx4])
    packed = jnp.where(iota_i < 4, res_j,
                       jnp.where(iota_i < 8,
                                 lax.bitcast_convert_type(e_perm, jnp.int32),
                                 lax.bitcast_convert_type(w_perm, jnp.int32)))
    res[...] = packed
    pltpu.sync_copy(res, out.at[wid])


def _tc_body(in_ref, oj_ref, oe_ref, ow_ref, bt_ref, bw_ref, bj_ref):
    rb = pl.program_id(0)
    kc = pl.program_id(1)
    nkc = pl.num_programs(1)

    @pl.when(kc == 0)
    def _():
        bt_ref[...] = jnp.full((TC_BR, 128), jnp.inf, jnp.float32)
        bw_ref[...] = jnp.ones((TC_BR, 128), jnp.float32)
        bj_ref[...] = jnp.zeros((TC_BR, 128), jnp.int32)

    rows = lax.broadcasted_iota(jnp.int32, (TC_BR, TC_BC), 0) + rb * TC_BR
    cols = lax.broadcasted_iota(jnp.int32, (TC_BR, TC_BC), 1) + kc * TC_BC
    cnt = lax.bitcast_convert_type(rows * N_COLS + cols, jnp.uint32)
    u = _uniform(_threefry_bits(cnt))
    ev = -jnp.log(u)
    wv = jnp.maximum(in_ref[...], jnp.float32(1e-30))

    bt = bt_ref[...]
    bw = bw_ref[...]
    bj = bj_ref[...]
    for s in range(TC_BC // 128):
        sl = (slice(None), slice(s * 128, (s + 1) * 128))
        es, ws = ev[sl], wv[sl]
        js = cols[sl]
        pred = es * bw < bt * ws
        bt = jnp.where(pred, es, bt)
        bw = jnp.where(pred, ws, bw)
        bj = jnp.where(pred, js, bj)
    bt_ref[...] = bt
    bw_ref[...] = bw
    bj_ref[...] = bj

    @pl.when(kc == nkc - 1)
    def _():
        q = bt / bw
        qmin = jnp.min(q, axis=1, keepdims=True)
        cand = jnp.where(q == qmin, bj, jnp.int32(2**31 - 1))
        jmin = jnp.min(cand, axis=1, keepdims=True)
        sel = cand == jmin
        oj_ref[...] = jmin
        oe_ref[...] = jnp.sum(jnp.where(sel, bt, 0.0), axis=1, keepdims=True)
        ow_ref[...] = jnp.sum(jnp.where(sel, bw, 0.0), axis=1, keepdims=True)


def _tc_sampler(inp_tc):
    grid = (N_ROWS // TC_BR, TC_COLS // TC_BC)
    return pl.pallas_call(
        _tc_body,
        grid=grid,
        in_specs=[pl.BlockSpec((TC_BR, TC_BC), lambda i, k: (i, k))],
        out_specs=[pl.BlockSpec((TC_BR, 1), lambda i, k: (i, 0))] * 3,
        out_shape=[
            jax.ShapeDtypeStruct((N_ROWS, 1), jnp.int32),
            jax.ShapeDtypeStruct((N_ROWS, 1), jnp.float32),
            jax.ShapeDtypeStruct((N_ROWS, 1), jnp.float32),
        ],
        scratch_shapes=[
            pltpu.VMEM((TC_BR, 128), jnp.float32),
            pltpu.VMEM((TC_BR, 128), jnp.float32),
            pltpu.VMEM((TC_BR, 128), jnp.int32),
        ],
        compiler_params=pltpu.CompilerParams(
            dimension_semantics=("arbitrary", "arbitrary")),
    )(inp_tc)


@jax.jit
def kernel(input):
    flat = input.reshape(-1)
    mesh = plsc.VectorSubcoreMesh(core_axis_name="c", subcore_axis_name="s")
    sc_sampler = pl.kernel(
        _sc_body,
        mesh=mesh,
        compiler_params=pltpu.CompilerParams(needs_layout_passes=False),
        out_type=jax.ShapeDtypeStruct((32, 16), jnp.int32),
        scratch_types=[
            pltpu.VMEM((SC_COLS,), jnp.float32),
            pltpu.VMEM((16,), jnp.int32),
            pltpu.VMEM((16,), jnp.float32),
            pltpu.VMEM((16,), jnp.float32),
            pltpu.VMEM((16,), jnp.int32),
        ],
    )
    sc_out = sc_sampler(flat)
    tc_j, tc_e, tc_w = _tc_sampler(input)

    sc_j = sc_out[:, 0:4].reshape(N_ROWS)
    sc_e = lax.bitcast_convert_type(sc_out[:, 4:8].reshape(N_ROWS),
                                    jnp.float32)
    sc_w = lax.bitcast_convert_type(sc_out[:, 8:12].reshape(N_ROWS),
                                    jnp.float32)
    tc_j = tc_j[:, 0]
    tc_e = tc_e[:, 0]
    tc_w = tc_w[:, 0]
    sc_wins = sc_e * tc_w < tc_e * sc_w
    return jnp.where(sc_wins, sc_j, tc_j).astype(jnp.int64)


# speedup vs baseline: 2.3309x; 2.1055x over previous
import functools

import numpy as np

import jax
import jax.numpy as jnp
from jax import lax
from jax.experimental import pallas as pl
from jax.experimental.pallas import tpu as pltpu
from jax.experimental.pallas import tpu_sc as plsc

N_ROWS = 128
N_COLS = 100000
TC_COLS = 81920
SC_COLS = N_COLS - TC_COLS
ROWS_PER_W = 4
UNROLL = 2
TC_BR = 8
TC_BC = 8192
TC_SUB = 2048

_U32 = jnp.uint32
_KS1 = 42
_KS2 = 0x1BD11BDA ^ 42


def _rotl(x, r):
    return lax.shift_left(x, _U32(r)) | lax.shift_right_logical(x, _U32(32 - r))


def _threefry_bits(cnt):
    x0 = jnp.zeros_like(cnt)
    x1 = cnt + _U32(_KS1)
    rots1 = (13, 15, 26, 6)
    rots2 = (17, 29, 16, 24)
    kinj = ((_KS1, _KS2 + 1), (_KS2, 0 + 2), (0, _KS1 + 3),
            (_KS1, _KS2 + 4), (_KS2, 0 + 5))
    for gi, rots in enumerate((rots1, rots2, rots1, rots2, rots1)):
        for r in rots:
            x0 = x0 + x1
            x1 = _rotl(x1, r)
            x1 = x1 ^ x0
        ka, kb = kinj[gi]
        x0 = x0 + _U32(ka)
        x1 = x1 + _U32(kb)
    return x0 ^ x1


_LN2_HI = np.float32(0.693359375)
_LN2_LO = np.float32(-2.12194440054e-4)
_SQRT2 = np.float32(1.4142135623730951)
_TINY = np.float32(1.1754943508222875e-38)


def _neg_log_poly(u):
    iv = lax.bitcast_convert_type(u, jnp.int32)
    e = lax.shift_right_arithmetic(iv, 23) - 127
    m = lax.bitcast_convert_type((iv & 0x7FFFFF) | 0x3F800000, jnp.float32)
    big = m > _SQRT2
    m = jnp.where(big, m * jnp.float32(0.5), m)
    e = jnp.where(big, e + 1, e)
    ef = e.astype(jnp.float32)
    x = m - jnp.float32(1.0)
    z = x * x
    y = jnp.float32(7.0376836292e-2)
    for c in (-1.1514610310e-1, 1.1676998740e-1, -1.2420140846e-1,
              1.4249322787e-1, -1.6668057665e-1, 2.0000714765e-1,
              -2.4999993993e-1, 3.3333331174e-1):
        y = y * x + jnp.float32(c)
    y = y * x * z
    y = y + ef * _LN2_LO
    y = y - jnp.float32(0.5) * z
    return -((x + y) + ef * _LN2_HI)


def _uniform(bits):
    fbits = lax.shift_right_logical(bits, _U32(9)) | _U32(0x3F800000)
    f = lax.bitcast_convert_type(fbits, jnp.float32) - jnp.float32(1.0)
    return jnp.maximum(f, _TINY)


def _sc_body(inp, out, buf, res, me, mw_r, mj_r):
    wid = lax.axis_index("s") * 2 + lax.axis_index("c")
    iota_i = lax.iota(jnp.int32, 16)
    iota_u = lax.convert_element_type(iota_i, jnp.uint32)
    res_j = jnp.zeros((16,), jnp.int32)
    res_e = jnp.zeros((16,), jnp.float32)
    res_w = jnp.zeros((16,), jnp.float32)

    for l in range(ROWS_PER_W):
        row = wid * ROWS_PER_W + l
        row_base = row * N_COLS + TC_COLS
        off = pl.multiple_of(row_base, 8)
        pltpu.sync_copy(inp.at[pl.ds(off, SC_COLS)], buf)

        def inner(t, carry, row_base=row_base):
            bt, bw, bj = carry
            j0 = t * (16 * UNROLL)
            for k in range(UNROLL):
                jk = j0 + 16 * k
                cbits = lax.convert_element_type(row_base + jk, jnp.uint32)
                cnt = cbits + iota_u
                ev = _neg_log_poly(_uniform(_threefry_bits(cnt)))
                wv = jnp.maximum(buf[pl.ds(jk, 16)], jnp.float32(1e-30))
                pred = ev * bw[k] < bt[k] * wv
                jv = (TC_COLS + jk) + iota_i
                bt[k] = jnp.where(pred, ev, bt[k])
                bw[k] = jnp.where(pred, wv, bw[k])
                bj[k] = jnp.where(pred, jv, bj[k])
            return bt, bw, bj

        init = ([jnp.full((16,), jnp.inf, jnp.float32)] * UNROLL,
                [jnp.ones((16,), jnp.float32)] * UNROLL,
                [jnp.zeros((16,), jnp.int32)] * UNROLL)
        bt, bw, bj = lax.fori_loop(0, SC_COLS // (16 * UNROLL), inner, init)

        mt, mw, mj = bt[0], bw[0], bj[0]
        for k in range(1, UNROLL):
            t1 = bt[k] * mw
            t2 = mt * bw[k]
            p = (t1 < t2) | ((t1 == t2) & (bj[k] < mj))
            mt = jnp.where(p, bt[k], mt)
            mw = jnp.where(p, bw[k], mw)
            mj = jnp.where(p, bj[k], mj)
        for d in (8, 4, 2, 1):
            me[...] = mt
            mw_r[...] = mw
            mj_r[...] = mj
            perm = iota_i ^ d
            e2 = plsc.load_gather(me, [perm])
            w2 = plsc.load_gather(mw_r, [perm])
            j2 = plsc.load_gather(mj_r, [perm])
            t1 = e2 * mw
            t2 = mt * w2
            p = (t1 < t2) | ((t1 == t2) & (j2 < mj))
            mt = jnp.where(p, e2, mt)
            mw = jnp.where(p, w2, mw)
            mj = jnp.where(p, j2, mj)
        res_j = jnp.where(iota_i == l, mj, res_j)
        res_e = jnp.where(iota_i == l, mt, res_e)
        res_w = jnp.where(iota_i == l, mw, res_w)

    me[...] = res_e
    mw_r[...] = res_w
    idx4 = iota_i & 3
    e_perm = plsc.load_gather(me, [idx4])
    w_perm = plsc.load_gather(mw_r, [idx4])
    packed = jnp.where(iota_i < 4, res_j,
                       jnp.where(iota_i < 8,
                                 lax.bitcast_convert_type(e_perm, jnp.int32),
                                 lax.bitcast_convert_type(w_perm, jnp.int32)))
    res[...] = packed
    pltpu.sync_copy(res, out.at[wid])


def _tc_body(in_ref, bt_ref, bw_ref, bj_ref):
    rb = pl.program_id(0)
    kc = pl.program_id(1)

    @pl.when(kc == 0)
    def _():
        bt_ref[...] = jnp.full((TC_BR, 128), jnp.inf, jnp.float32)
        bw_ref[...] = jnp.ones((TC_BR, 128), jnp.float32)
        bj_ref[...] = jnp.zeros((TC_BR, 128), jnp.int32)

    bt = bt_ref[...]
    bw = bw_ref[...]
    bj = bj_ref[...]
    for h in range(TC_BC // TC_SUB):
        rows = lax.broadcasted_iota(jnp.int32, (TC_BR, TC_SUB), 0) + rb * TC_BR
        cols = (lax.broadcasted_iota(jnp.int32, (TC_BR, TC_SUB), 1)
                + (kc * TC_BC + h * TC_SUB))
        cnt = lax.bitcast_convert_type(rows * N_COLS + cols, jnp.uint32)
        ev = _neg_log_poly(_uniform(_threefry_bits(cnt)))
        wv = jnp.maximum(in_ref[:, h * TC_SUB:(h + 1) * TC_SUB],
                         jnp.float32(1e-30))
        for s in range(TC_SUB // 128):
            sl = (slice(None), slice(s * 128, (s + 1) * 128))
            es, ws = ev[sl], wv[sl]
            js = cols[sl]
            pred = es * bw < bt * ws
            bt = jnp.where(pred, es, bt)
            bw = jnp.where(pred, ws, bw)
            bj = jnp.where(pred, js, bj)
    bt_ref[...] = bt
    bw_ref[...] = bw
    bj_ref[...] = bj


def _tc_merge_body(bt_ref, bw_ref, bj_ref, oj_ref, oe_ref, ow_ref):
    bt = bt_ref[...]
    bw = bw_ref[...]
    bj = bj_ref[...]
    q = bt / bw
    qmin = jnp.min(q, axis=1, keepdims=True)
    cand = jnp.where(q == qmin, bj, jnp.int32(2**31 - 1))
    jmin = jnp.min(cand, axis=1, keepdims=True)
    sel = cand == jmin
    oj_ref[...] = jmin
    oe_ref[...] = jnp.sum(jnp.where(sel, bt, 0.0), axis=1, keepdims=True)
    ow_ref[...] = jnp.sum(jnp.where(sel, bw, 0.0), axis=1, keepdims=True)


def _tc_sampler(inp_tc):
    grid = (N_ROWS // TC_BR, TC_COLS // TC_BC)
    bt, bw, bj = pl.pallas_call(
        _tc_body,
        grid=grid,
        in_specs=[pl.BlockSpec((TC_BR, TC_BC), lambda i, k: (i, k))],
        out_specs=[pl.BlockSpec((TC_BR, 128), lambda i, k: (i, 0))] * 3,
        out_shape=[
            jax.ShapeDtypeStruct((N_ROWS, 128), jnp.float32),
            jax.ShapeDtypeStruct((N_ROWS, 128), jnp.float32),
            jax.ShapeDtypeStruct((N_ROWS, 128), jnp.int32),
        ],
        compiler_params=pltpu.CompilerParams(
            dimension_semantics=("arbitrary", "arbitrary")),
    )(inp_tc)
    return pl.pallas_call(
        _tc_merge_body,
        out_shape=[
            jax.ShapeDtypeStruct((N_ROWS, 1), jnp.int32),
            jax.ShapeDtypeStruct((N_ROWS, 1), jnp.float32),
            jax.ShapeDtypeStruct((N_ROWS, 1), jnp.float32),
        ],
    )(bt, bw, bj)


@jax.jit
def kernel(input):
    flat = input.reshape(-1)
    mesh = plsc.VectorSubcoreMesh(core_axis_name="c", subcore_axis_name="s")
    sc_sampler = pl.kernel(
        _sc_body,
        mesh=mesh,
        compiler_params=pltpu.CompilerParams(needs_layout_passes=False),
        out_type=jax.ShapeDtypeStruct((32, 16), jnp.int32),
        scratch_types=[
            pltpu.VMEM((SC_COLS,), jnp.float32),
            pltpu.VMEM((16,), jnp.int32),
            pltpu.VMEM((16,), jnp.float32),
            pltpu.VMEM((16,), jnp.float32),
            pltpu.VMEM((16,), jnp.int32),
        ],
    )
    sc_out = sc_sampler(flat)
    tc_j, tc_e, tc_w = _tc_sampler(input)

    sc_j = sc_out[:, 0:4].reshape(N_ROWS)
    sc_e = lax.bitcast_convert_type(sc_out[:, 4:8].reshape(N_ROWS),
                                    jnp.float32)
    sc_w = lax.bitcast_convert_type(sc_out[:, 8:12].reshape(N_ROWS),
                                    jnp.float32)
    tc_j = tc_j[:, 0]
    tc_e = tc_e[:, 0]
    tc_w = tc_w[:, 0]
    sc_wins = sc_e * tc_w < tc_e * sc_w
    return jnp.where(sc_wins, sc_j, tc_j).astype(jnp.int64)


# speedup vs baseline: 2.3310x; 1.0000x over previous
import functools

import numpy as np

import jax
import jax.numpy as jnp
from jax import lax
from jax.experimental import pallas as pl
from jax.experimental.pallas import tpu as pltpu
from jax.experimental.pallas import tpu_sc as plsc

N_ROWS = 128
N_COLS = 100000
TC_COLS = 81920
SC_COLS = N_COLS - TC_COLS
ROWS_PER_W = 4
UNROLL = 2
TC_BR = 8
TC_BC = 8192
TC_SUB = 2048

_U32 = jnp.uint32
_KS1 = 42
_KS2 = 0x1BD11BDA ^ 42


def _rotl(x, r):
    return lax.shift_left(x, _U32(r)) | lax.shift_right_logical(x, _U32(32 - r))


def _threefry_bits(cnt):
    x0 = jnp.zeros_like(cnt)
    x1 = cnt + _U32(_KS1)
    rots1 = (13, 15, 26, 6)
    rots2 = (17, 29, 16, 24)
    kinj = ((_KS1, _KS2 + 1), (_KS2, 0 + 2), (0, _KS1 + 3),
            (_KS1, _KS2 + 4), (_KS2, 0 + 5))
    for gi, rots in enumerate((rots1, rots2, rots1, rots2, rots1)):
        for r in rots:
            x0 = x0 + x1
            x1 = _rotl(x1, r)
            x1 = x1 ^ x0
        ka, kb = kinj[gi]
        x0 = x0 + _U32(ka)
        x1 = x1 + _U32(kb)
    return x0 ^ x1


_LN2_HI = np.float32(0.693359375)
_LN2_LO = np.float32(-2.12194440054e-4)
_SQRT2 = np.float32(1.4142135623730951)
_TINY = np.float32(1.1754943508222875e-38)


def _neg_log_poly(u):
    iv = lax.bitcast_convert_type(u, jnp.int32)
    e = lax.shift_right_arithmetic(iv, 23) - 127
    m = lax.bitcast_convert_type((iv & 0x7FFFFF) | 0x3F800000, jnp.float32)
    big = m > _SQRT2
    m = jnp.where(big, m * jnp.float32(0.5), m)
    e = jnp.where(big, e + 1, e)
    ef = e.astype(jnp.float32)
    x = m - jnp.float32(1.0)
    z = x * x
    y = jnp.float32(7.0376836292e-2)
    for c in (-1.1514610310e-1, 1.1676998740e-1, -1.2420140846e-1,
              1.4249322787e-1, -1.6668057665e-1, 2.0000714765e-1,
              -2.4999993993e-1, 3.3333331174e-1):
        y = y * x + jnp.float32(c)
    y = y * x * z
    y = y + ef * _LN2_LO
    y = y - jnp.float32(0.5) * z
    return -((x + y) + ef * _LN2_HI)


def _uniform(bits):
    fbits = lax.shift_right_logical(bits, _U32(9)) | _U32(0x3F800000)
    f = lax.bitcast_convert_type(fbits, jnp.float32) - jnp.float32(1.0)
    return jnp.maximum(f, _TINY)


def _sc_body(inp, out, buf, res, me, mw_r, mj_r):
    wid = lax.axis_index("s") * 2 + lax.axis_index("c")
    iota_i = lax.iota(jnp.int32, 16)
    iota_u = lax.convert_element_type(iota_i, jnp.uint32)
    res_j = jnp.zeros((16,), jnp.int32)
    res_e = jnp.zeros((16,), jnp.float32)
    res_w = jnp.zeros((16,), jnp.float32)

    for l in range(ROWS_PER_W):
        row = wid * ROWS_PER_W + l
        row_base = row * N_COLS + TC_COLS
        off = pl.multiple_of(row_base, 8)
        pltpu.sync_copy(inp.at[pl.ds(off, SC_COLS)], buf)

        def inner(t, carry, row_base=row_base):
            bt, bw, bj = carry
            j0 = t * (16 * UNROLL)
            for k in range(UNROLL):
                jk = j0 + 16 * k
                cbits = lax.convert_element_type(row_base + jk, jnp.uint32)
                cnt = cbits + iota_u
                ev = _neg_log_poly(_uniform(_threefry_bits(cnt)))
                wv = jnp.maximum(buf[pl.ds(jk, 16)], jnp.float32(1e-30))
                pred = ev * bw[k] < bt[k] * wv
                jv = (TC_COLS + jk) + iota_i
                bt[k] = jnp.where(pred, ev, bt[k])
                bw[k] = jnp.where(pred, wv, bw[k])
                bj[k] = jnp.where(pred, jv, bj[k])
            return bt, bw, bj

        init = ([jnp.full((16,), jnp.inf, jnp.float32)] * UNROLL,
                [jnp.ones((16,), jnp.float32)] * UNROLL,
                [jnp.zeros((16,), jnp.int32)] * UNROLL)
        bt, bw, bj = lax.fori_loop(0, SC_COLS // (16 * UNROLL), inner, init)

        mt, mw, mj = bt[0], bw[0], bj[0]
        for k in range(1, UNROLL):
            t1 = bt[k] * mw
            t2 = mt * bw[k]
            p = (t1 < t2) | ((t1 == t2) & (bj[k] < mj))
            mt = jnp.where(p, bt[k], mt)
            mw = jnp.where(p, bw[k], mw)
            mj = jnp.where(p, bj[k], mj)
        for d in (8, 4, 2, 1):
            me[...] = mt
            mw_r[...] = mw
            mj_r[...] = mj
            perm = iota_i ^ d
            e2 = plsc.load_gather(me, [perm])
            w2 = plsc.load_gather(mw_r, [perm])
            j2 = plsc.load_gather(mj_r, [perm])
            t1 = e2 * mw
            t2 = mt * w2
            p = (t1 < t2) | ((t1 == t2) & (j2 < mj))
            mt = jnp.where(p, e2, mt)
            mw = jnp.where(p, w2, mw)
            mj = jnp.where(p, j2, mj)
        res_j = jnp.where(iota_i == l, mj, res_j)
        res_e = jnp.where(iota_i == l, mt, res_e)
        res_w = jnp.where(iota_i == l, mw, res_w)

    me[...] = res_e
    mw_r[...] = res_w
    idx4 = iota_i & 3
    e_perm = plsc.load_gather(me, [idx4])
    w_perm = plsc.load_gather(mw_r, [idx4])
    packed = jnp.where(iota_i < 4, res_j,
                       jnp.where(iota_i < 8,
                                 lax.bitcast_convert_type(e_perm, jnp.int32),
                                 lax.bitcast_convert_type(w_perm, jnp.int32)))
    res[...] = packed
    pltpu.sync_copy(res, out.at[wid])


def _tc_body(in_ref, bt_ref, bw_ref, bj_ref):
    rb = pl.program_id(0)
    kc = pl.program_id(1)

    @pl.when(kc == 0)
    def _():
        bt_ref[...] = jnp.full((TC_BR, 128), jnp.inf, jnp.float32)
        bw_ref[...] = jnp.ones((TC_BR, 128), jnp.float32)
        bj_ref[...] = jnp.zeros((TC_BR, 128), jnp.int32)

    bt = bt_ref[...]
    bw = bw_ref[...]
    bj = bj_ref[...]
    for h in range(TC_BC // TC_SUB):
        rows = lax.broadcasted_iota(jnp.int32, (TC_BR, TC_SUB), 0) + rb * TC_BR
        cols = (lax.broadcasted_iota(jnp.int32, (TC_BR, TC_SUB), 1)
                + (kc * TC_BC + h * TC_SUB))
        cnt = lax.bitcast_convert_type(rows * N_COLS + cols, jnp.uint32)
        ev = _neg_log_poly(_uniform(_threefry_bits(cnt)))
        wv = jnp.maximum(in_ref[:, h * TC_SUB:(h + 1) * TC_SUB],
                         jnp.float32(1e-30))
        for s in range(TC_SUB // 128):
            sl = (slice(None), slice(s * 128, (s + 1) * 128))
            es, ws = ev[sl], wv[sl]
            js = cols[sl]
            pred = es * bw < bt * ws
            bt = jnp.where(pred, es, bt)
            bw = jnp.where(pred, ws, bw)
            bj = jnp.where(pred, js, bj)
    bt_ref[...] = bt
    bw_ref[...] = bw
    bj_ref[...] = bj


def _tc_merge_body(bt_ref, bw_ref, bj_ref, oj_ref, oe_ref, ow_ref):
    bt = bt_ref[...]
    bw = bw_ref[...]
    bj = bj_ref[...]
    q = bt / bw
    qmin = jnp.min(q, axis=1, keepdims=True)
    cand = jnp.where(q == qmin, bj, jnp.int32(2**31 - 1))
    jmin = jnp.min(cand, axis=1, keepdims=True)
    sel = cand == jmin
    oj_ref[...] = jmin
    oe_ref[...] = jnp.sum(jnp.where(sel, bt, 0.0), axis=1, keepdims=True)
    ow_ref[...] = jnp.sum(jnp.where(sel, bw, 0.0), axis=1, keepdims=True)


def _tc_sampler(inp_tc):
    grid = (N_ROWS // TC_BR, TC_COLS // TC_BC)
    bt, bw, bj = pl.pallas_call(
        _tc_body,
        grid=grid,
        in_specs=[pl.BlockSpec((TC_BR, TC_BC), lambda i, k: (i, k))],
        out_specs=[pl.BlockSpec((TC_BR, 128), lambda i, k: (i, 0))] * 3,
        out_shape=[
            jax.ShapeDtypeStruct((N_ROWS, 128), jnp.float32),
            jax.ShapeDtypeStruct((N_ROWS, 128), jnp.float32),
            jax.ShapeDtypeStruct((N_ROWS, 128), jnp.int32),
        ],
        compiler_params=pltpu.CompilerParams(
            dimension_semantics=("arbitrary", "arbitrary")),
        cost_estimate=pl.CostEstimate(
            flops=3_200_000_000, bytes_accessed=45_000_000,
            transcendentals=0),
    )(inp_tc)
    return pl.pallas_call(
        _tc_merge_body,
        out_shape=[
            jax.ShapeDtypeStruct((N_ROWS, 1), jnp.int32),
            jax.ShapeDtypeStruct((N_ROWS, 1), jnp.float32),
            jax.ShapeDtypeStruct((N_ROWS, 1), jnp.float32),
        ],
    )(bt, bw, bj)


@jax.jit
def kernel(input):
    flat = input.reshape(-1)
    mesh = plsc.VectorSubcoreMesh(core_axis_name="c", subcore_axis_name="s")
    sc_sampler = pl.kernel(
        _sc_body,
        mesh=mesh,
        compiler_params=pltpu.CompilerParams(needs_layout_passes=False),
        out_type=jax.ShapeDtypeStruct((32, 16), jnp.int32),
        scratch_types=[
            pltpu.VMEM((SC_COLS,), jnp.float32),
            pltpu.VMEM((16,), jnp.int32),
            pltpu.VMEM((16,), jnp.float32),
            pltpu.VMEM((16,), jnp.float32),
            pltpu.VMEM((16,), jnp.int32),
        ],
    )
    sc_out = sc_sampler(flat)
    tc_j, tc_e, tc_w = _tc_sampler(input)

    sc_j = sc_out[:, 0:4].reshape(N_ROWS)
    sc_e = lax.bitcast_convert_type(sc_out[:, 4:8].reshape(N_ROWS),
                                    jnp.float32)
    sc_w = lax.bitcast_convert_type(sc_out[:, 8:12].reshape(N_ROWS),
                                    jnp.float32)
    tc_j = tc_j[:, 0]
    tc_e = tc_e[:, 0]
    tc_w = tc_w[:, 0]
    sc_wins = sc_e * tc_w < tc_e * sc_w
    return jnp.where(sc_wins, sc_j, tc_j).astype(jnp.int64)
